# baseline (speedup 1.0000x reference)
import functools

import jax
import jax.numpy as jnp
from jax import lax
from jax.experimental import pallas as pl
from jax.experimental.pallas import tpu as pltpu
from jax.experimental.pallas import tpu_sc as plsc

B = 1024
D = 64
K = 100000
NN = 5
KC = 2176
NCH = 46
CAND = NCH * NN
CPAD = 240
NW = 32
QPW = B // NW


BB = 256
NB = B // BB
S = 128
NSTR = KC // S


def _stage1(x_ref, q2_ref, db2_ref, xn_ref, vals_ref, idx_ref, d_ref):
    j = pl.program_id(0)
    i = pl.program_id(1)
    dots = lax.dot_general(x_ref[...], xn_ref[...], (((1,), (1,)), ((), ())),
                           preferred_element_type=jnp.float32,
                           precision=lax.Precision.DEFAULT)
    d_ref[...] = (q2_ref[...] + db2_ref[...]) - 2.0 * dots
    big = jnp.int32(2 ** 30)

    def pass_body(t, carry):
        m_prev, am_prev, mv, iv = carry

        def strip(s, c2):
            m, am = c2
            dc = d_ref[:, pl.ds(pl.multiple_of(s * S, S), S)]
            col = lax.broadcasted_iota(jnp.int32, (BB, S), 1) + s * S
            ok = (dc > m_prev[:, None]) | (
                (dc == m_prev[:, None]) & (col > am_prev[:, None]))
            dm = jnp.where(ok, dc, jnp.inf)
            lm = jnp.min(dm, axis=1)
            lam = jnp.min(jnp.where(dm == lm[:, None], col, big), axis=1)
            better = (lm < m) | ((lm == m) & (lam < am))
            return jnp.where(better, lm, m), jnp.where(better, lam, am)

        m, am = lax.fori_loop(
            0, NSTR, strip,
            (jnp.full((BB,), jnp.inf, jnp.float32),
             jnp.full((BB,), big)))
        mv = tuple(jnp.where(t == tt, m, mv[tt]) for tt in range(NN))
        iv = tuple(jnp.where(t == tt, am, iv[tt]) for tt in range(NN))
        return m, am, mv, iv

    zf = jnp.zeros((BB,), jnp.float32)
    zi = jnp.zeros((BB,), jnp.int32)
    _, _, mv, iv = lax.fori_loop(
        0, NN, pass_body,
        (jnp.full((BB,), -jnp.inf, jnp.float32),
         jnp.full((BB,), jnp.int32(-1)), (zf,) * NN, (zi,) * NN))
    for tt in range(NN):
        vals_ref[0, tt, pl.ds(i * BB, BB)] = mv[tt]
        idx_ref[0, tt, pl.ds(i * BB, BB)] = iv[tt] + j * KC


def _stage2(vals_hbm, idx_hbm, y_hbm, out_hbm, vv, iv, yg, ov, sem):
    wid = lax.axis_index("s") * 2 + lax.axis_index("c")
    for g in range(QPW // 16):
        gi = wid * (QPW // 16) + g
        pltpu.sync_copy(vals_hbm.at[pl.ds(gi * CPAD * 16, CPAD * 16)], vv)
        pltpu.sync_copy(idx_hbm.at[pl.ds(gi * CPAD * 16, CPAD * 16)], iv)
        carry0 = (jnp.full((16,), jnp.inf, jnp.float32),) * NN + (
            jnp.zeros((16,), jnp.int32),) * NN

        def body(c, carry):
            bv = list(carry[:NN])
            bi = list(carry[NN:])
            v = vv[pl.ds(c * 16, 16)]
            ix = iv[pl.ds(c * 16, 16)]
            for t in range(NN):
                sw = v < bv[t]
                nv = jnp.where(sw, v, bv[t])
                ni = jnp.where(sw, ix, bi[t])
                v = jnp.where(sw, bv[t], v)
                ix = jnp.where(sw, bi[t], ix)
                bv[t] = nv
                bi[t] = ni
            return tuple(bv) + tuple(bi)

        carry = lax.fori_loop(0, CPAD, body, carry0)
        for t in range(NN):
            pltpu.async_copy(y_hbm.at[carry[NN + t]], yg.at[t], sem).wait()
        s = yg[0, :]
        for t in range(1, NN):
            s = s + yg[t, :]
        ov[...] = s * jnp.float32(1.0 / NN)
        pltpu.sync_copy(ov, out_hbm.at[pl.ds(gi * 16, 16)])


def _make_stage2():
    mesh = plsc.VectorSubcoreMesh(core_axis_name="c", subcore_axis_name="s")
    return pl.kernel(
        _stage2,
        mesh=mesh,
        out_type=jax.ShapeDtypeStruct((B,), jnp.float32),
        scratch_types=[
            pltpu.VMEM((CPAD * 16,), jnp.float32),
            pltpu.VMEM((CPAD * 16,), jnp.int32),
            pltpu.VMEM((NN, 16), jnp.float32),
            pltpu.VMEM((16,), jnp.float32),
            pltpu.SemaphoreType.DMA,
        ],
    )


def kernel(feats, feat_means, feat_stds, Xn, y_arr):
    x = (feats - feat_means) / feat_stds
    q2 = jnp.sum(x * x, axis=1, keepdims=True)
    db2 = jnp.sum(Xn * Xn, axis=1)[None, :]
    kpad = NCH * KC - K
    xn_pad = jnp.pad(Xn, ((0, kpad), (0, 0)))
    db2_pad = jnp.pad(db2, ((0, 0), (0, kpad)),
                      constant_values=jnp.inf)
    vals_t, idx_t = pl.pallas_call(
        _stage1,
        grid=(NCH, NB),
        in_specs=[
            pl.BlockSpec((BB, D), lambda j, i: (i, 0)),
            pl.BlockSpec((BB, 1), lambda j, i: (i, 0)),
            pl.BlockSpec((1, KC), lambda j, i: (0, j)),
            pl.BlockSpec((KC, D), lambda j, i: (j, 0)),
        ],
        out_specs=[
            pl.BlockSpec((1, NN, B), lambda j, i: (j, 0, 0)),
            pl.BlockSpec((1, NN, B), lambda j, i: (j, 0, 0)),
        ],
        out_shape=[
            jax.ShapeDtypeStruct((NCH, NN, B), jnp.float32),
            jax.ShapeDtypeStruct((NCH, NN, B), jnp.int32),
        ],
        scratch_shapes=[pltpu.VMEM((BB, KC), jnp.float32)],
    )(x, q2, db2_pad, xn_pad)

    vals2 = jnp.concatenate(
        [vals_t.reshape(CAND, B),
         jnp.full((CPAD - CAND, B), jnp.inf, jnp.float32)], axis=0)
    idx2 = jnp.concatenate(
        [idx_t.reshape(CAND, B),
         jnp.zeros((CPAD - CAND, B), jnp.int32)], axis=0)
    vflat = vals2.reshape(CPAD, B // 16, 16).transpose(1, 0, 2).reshape(-1)
    iflat = idx2.reshape(CPAD, B // 16, 16).transpose(1, 0, 2).reshape(-1)
    preds = _make_stage2()(vflat, iflat, y_arr)
    return preds[:, None]


# speedup vs baseline: 3.2848x; 3.2848x over previous
import functools

import jax
import jax.numpy as jnp
from jax import lax
from jax.experimental import pallas as pl
from jax.experimental.pallas import tpu as pltpu
from jax.experimental.pallas import tpu_sc as plsc

B = 1024
D = 64
K = 100000
NN = 5
KC = 2176
NCH = 46
CAND = NCH * NN
CPAD = 240
NW = 32
QPW = B // NW


BB = 512
NB = B // BB
S = 128
NSTR = KC // S


def _stage1(x_ref, q2_ref, db2_ref, xn_ref, vals_ref, idx_ref, d_ref,
            best_ref, cbest_ref):
    j = pl.program_id(0)
    i = pl.program_id(1)
    dots = lax.dot_general(x_ref[...], xn_ref[...], (((1,), (1,)), ((), ())),
                           preferred_element_type=jnp.float32,
                           precision=lax.Precision.DEFAULT)
    d_ref[...] = (q2_ref[...] + db2_ref[...]) - 2.0 * dots
    big = jnp.int32(2 ** 30)
    inf = jnp.float32(jnp.inf)

    def pass_body(t, carry):
        m_prev, am_prev, mv, iv = carry
        mp = m_prev[:, None]
        ap = am_prev[:, None]
        best_ref[...] = jnp.full((BB, S), inf, jnp.float32)
        cbest_ref[...] = jnp.full((BB, S), big, jnp.int32)

        def strip(s, _):
            dc = d_ref[:, pl.ds(pl.multiple_of(s * S, S), S)]
            col = lax.broadcasted_iota(jnp.int32, (BB, S), 1) + s * S
            ok = (dc > mp) | ((dc == mp) & (col > ap))
            upd = ok & (dc < best_ref[...])
            best_ref[...] = jnp.where(upd, dc, best_ref[...])
            cbest_ref[...] = jnp.where(upd, col, cbest_ref[...])
            return 0

        lax.fori_loop(0, NSTR, strip, 0)
        best = best_ref[...]
        m = jnp.min(best, axis=1)
        am = jnp.min(jnp.where(best == m[:, None], cbest_ref[...], big),
                     axis=1)
        mv = tuple(jnp.where(t == tt, m, mv[tt]) for tt in range(NN))
        iv = tuple(jnp.where(t == tt, am, iv[tt]) for tt in range(NN))
        return m, am, mv, iv

    zf = jnp.zeros((BB,), jnp.float32)
    zi = jnp.zeros((BB,), jnp.int32)
    _, _, mv, iv = lax.fori_loop(
        0, NN, pass_body,
        (jnp.full((BB,), -inf, jnp.float32),
         jnp.full((BB,), jnp.int32(-1)), (zf,) * NN, (zi,) * NN))
    for tt in range(NN):
        vals_ref[0, tt, pl.ds(i * BB, BB)] = mv[tt]
        idx_ref[0, tt, pl.ds(i * BB, BB)] = iv[tt] + j * KC


def _stage2(vals_hbm, idx_hbm, y_hbm, out_hbm, vv, iv, yg, ov, sem):
    wid = lax.axis_index("s") * 2 + lax.axis_index("c")
    for g in range(QPW // 16):
        gi = wid * (QPW // 16) + g
        pltpu.sync_copy(vals_hbm.at[pl.ds(gi * CPAD * 16, CPAD * 16)], vv)
        pltpu.sync_copy(idx_hbm.at[pl.ds(gi * CPAD * 16, CPAD * 16)], iv)
        carry0 = (jnp.full((16,), jnp.inf, jnp.float32),) * NN + (
            jnp.zeros((16,), jnp.int32),) * NN

        def body(c, carry):
            bv = list(carry[:NN])
            bi = list(carry[NN:])
            v = vv[pl.ds(c * 16, 16)]
            ix = iv[pl.ds(c * 16, 16)]
            for t in range(NN):
                sw = v < bv[t]
                nv = jnp.where(sw, v, bv[t])
                ni = jnp.where(sw, ix, bi[t])
                v = jnp.where(sw, bv[t], v)
                ix = jnp.where(sw, bi[t], ix)
                bv[t] = nv
                bi[t] = ni
            return tuple(bv) + tuple(bi)

        carry = lax.fori_loop(0, CPAD, body, carry0)
        for t in range(NN):
            pltpu.async_copy(y_hbm.at[carry[NN + t]], yg.at[t], sem).wait()
        s = yg[0, :]
        for t in range(1, NN):
            s = s + yg[t, :]
        ov[...] = s * jnp.float32(1.0 / NN)
        pltpu.sync_copy(ov, out_hbm.at[pl.ds(gi * 16, 16)])


def _make_stage2():
    mesh = plsc.VectorSubcoreMesh(core_axis_name="c", subcore_axis_name="s")
    return pl.kernel(
        _stage2,
        mesh=mesh,
        out_type=jax.ShapeDtypeStruct((B,), jnp.float32),
        scratch_types=[
            pltpu.VMEM((CPAD * 16,), jnp.float32),
            pltpu.VMEM((CPAD * 16,), jnp.int32),
            pltpu.VMEM((NN, 16), jnp.float32),
            pltpu.VMEM((16,), jnp.float32),
            pltpu.SemaphoreType.DMA,
        ],
    )


def kernel(feats, feat_means, feat_stds, Xn, y_arr):
    x = (feats - feat_means) / feat_stds
    q2 = jnp.sum(x * x, axis=1, keepdims=True)
    db2 = jnp.sum(Xn * Xn, axis=1)[None, :]
    kpad = NCH * KC - K
    xn_pad = jnp.pad(Xn, ((0, kpad), (0, 0)))
    db2_pad = jnp.pad(db2, ((0, 0), (0, kpad)),
                      constant_values=jnp.inf)
    vals_t, idx_t = pl.pallas_call(
        _stage1,
        grid=(NCH, NB),
        in_specs=[
            pl.BlockSpec((BB, D), lambda j, i: (i, 0)),
            pl.BlockSpec((BB, 1), lambda j, i: (i, 0)),
            pl.BlockSpec((1, KC), lambda j, i: (0, j)),
            pl.BlockSpec((KC, D), lambda j, i: (j, 0)),
        ],
        out_specs=[
            pl.BlockSpec((1, NN, B), lambda j, i: (j, 0, 0)),
            pl.BlockSpec((1, NN, B), lambda j, i: (j, 0, 0)),
        ],
        out_shape=[
            jax.ShapeDtypeStruct((NCH, NN, B), jnp.float32),
            jax.ShapeDtypeStruct((NCH, NN, B), jnp.int32),
        ],
        scratch_shapes=[pltpu.VMEM((BB, KC), jnp.float32),
                        pltpu.VMEM((BB, S), jnp.float32),
                        pltpu.VMEM((BB, S), jnp.int32)],
    )(x, q2, db2_pad, xn_pad)

    vals2 = jnp.concatenate(
        [vals_t.reshape(CAND, B),
         jnp.full((CPAD - CAND, B), jnp.inf, jnp.float32)], axis=0)
    idx2 = jnp.concatenate(
        [idx_t.reshape(CAND, B),
         jnp.zeros((CPAD - CAND, B), jnp.int32)], axis=0)
    vflat = vals2.reshape(CPAD, B // 16, 16).transpose(1, 0, 2).reshape(-1)
    iflat = idx2.reshape(CPAD, B // 16, 16).transpose(1, 0, 2).reshape(-1)
    preds = _make_stage2()(vflat, iflat, y_arr)
    return preds[:, None]


# speedup vs baseline: 4.0255x; 1.2255x over previous
import functools

import jax
import jax.numpy as jnp
from jax import lax
from jax.experimental import pallas as pl
from jax.experimental.pallas import tpu as pltpu
from jax.experimental.pallas import tpu_sc as plsc

B = 1024
D = 64
K = 100000
NN = 5
KC = 4352
NCH = 23
CAND = NCH * NN
CPAD = 128
NW = 32
QPW = B // NW


BB = 512
NB = B // BB
S = 128
NSTR = KC // S


def _stage1(x_ref, q2_ref, db2_ref, xn_ref, vals_ref, idx_ref, d_ref,
            best_ref, cbest_ref):
    j = pl.program_id(0)
    i = pl.program_id(1)
    dots = lax.dot_general(x_ref[...], xn_ref[...], (((1,), (1,)), ((), ())),
                           preferred_element_type=jnp.float32,
                           precision=lax.Precision.DEFAULT)
    d_ref[...] = (q2_ref[...] + db2_ref[...]) - 2.0 * dots
    big = jnp.int32(2 ** 30)
    inf = jnp.float32(jnp.inf)

    def pass_body(t, carry):
        m_prev, am_prev, mv, iv = carry
        mp = m_prev[:, None]
        ap = am_prev[:, None]
        best_ref[...] = jnp.full((BB, S), inf, jnp.float32)
        cbest_ref[...] = jnp.full((BB, S), big, jnp.int32)

        def strip(s, _):
            dc = d_ref[:, pl.ds(pl.multiple_of(s * S, S), S)]
            col = lax.broadcasted_iota(jnp.int32, (BB, S), 1) + s * S
            ok = (dc > mp) | ((dc == mp) & (col > ap))
            upd = ok & (dc < best_ref[...])
            best_ref[...] = jnp.where(upd, dc, best_ref[...])
            cbest_ref[...] = jnp.where(upd, col, cbest_ref[...])
            return 0

        lax.fori_loop(0, NSTR, strip, 0)
        best = best_ref[...]
        m = jnp.min(best, axis=1)
        am = jnp.min(jnp.where(best == m[:, None], cbest_ref[...], big),
                     axis=1)
        mv = tuple(jnp.where(t == tt, m, mv[tt]) for tt in range(NN))
        iv = tuple(jnp.where(t == tt, am, iv[tt]) for tt in range(NN))
        return m, am, mv, iv

    zf = jnp.zeros((BB,), jnp.float32)
    zi = jnp.zeros((BB,), jnp.int32)
    _, _, mv, iv = lax.fori_loop(
        0, NN, pass_body,
        (jnp.full((BB,), -inf, jnp.float32),
         jnp.full((BB,), jnp.int32(-1)), (zf,) * NN, (zi,) * NN))
    for tt in range(NN):
        vals_ref[0, tt, pl.ds(i * BB, BB)] = mv[tt]
        idx_ref[0, tt, pl.ds(i * BB, BB)] = iv[tt] + j * KC


def _stage2(vals_hbm, idx_hbm, y_hbm, out_hbm, vv, iv, yg, ov, sem):
    wid = lax.axis_index("s") * 2 + lax.axis_index("c")
    for g in range(QPW // 16):
        gi = wid * (QPW // 16) + g
        pltpu.sync_copy(vals_hbm.at[pl.ds(gi * CPAD * 16, CPAD * 16)], vv)
        pltpu.sync_copy(idx_hbm.at[pl.ds(gi * CPAD * 16, CPAD * 16)], iv)
        carry0 = (jnp.full((16,), jnp.inf, jnp.float32),) * NN + (
            jnp.zeros((16,), jnp.int32),) * NN

        def body(c, carry):
            bv = list(carry[:NN])
            bi = list(carry[NN:])
            v = vv[pl.ds(c * 16, 16)]
            ix = iv[pl.ds(c * 16, 16)]
            for t in range(NN):
                sw = v < bv[t]
                nv = jnp.where(sw, v, bv[t])
                ni = jnp.where(sw, ix, bi[t])
                v = jnp.where(sw, bv[t], v)
                ix = jnp.where(sw, bi[t], ix)
                bv[t] = nv
                bi[t] = ni
            return tuple(bv) + tuple(bi)

        carry = lax.fori_loop(0, CPAD, body, carry0)
        for t in range(NN):
            pltpu.async_copy(y_hbm.at[carry[NN + t]], yg.at[t], sem).wait()
        s = yg[0, :]
        for t in range(1, NN):
            s = s + yg[t, :]
        ov[...] = s * jnp.float32(1.0 / NN)
        pltpu.sync_copy(ov, out_hbm.at[pl.ds(gi * 16, 16)])


def _make_stage2():
    mesh = plsc.VectorSubcoreMesh(core_axis_name="c", subcore_axis_name="s")
    return pl.kernel(
        _stage2,
        mesh=mesh,
        out_type=jax.ShapeDtypeStruct((B,), jnp.float32),
        scratch_types=[
            pltpu.VMEM((CPAD * 16,), jnp.float32),
            pltpu.VMEM((CPAD * 16,), jnp.int32),
            pltpu.VMEM((NN, 16), jnp.float32),
            pltpu.VMEM((16,), jnp.float32),
            pltpu.SemaphoreType.DMA,
        ],
    )


def kernel(feats, feat_means, feat_stds, Xn, y_arr):
    x = (feats - feat_means) / feat_stds
    q2 = jnp.sum(x * x, axis=1, keepdims=True)
    db2 = jnp.sum(Xn * Xn, axis=1)[None, :]
    kpad = NCH * KC - K
    xn_pad = jnp.pad(Xn, ((0, kpad), (0, 0)))
    db2_pad = jnp.pad(db2, ((0, 0), (0, kpad)),
                      constant_values=jnp.inf)
    vals_t, idx_t = pl.pallas_call(
        _stage1,
        grid=(NCH, NB),
        in_specs=[
            pl.BlockSpec((BB, D), lambda j, i: (i, 0)),
            pl.BlockSpec((BB, 1), lambda j, i: (i, 0)),
            pl.BlockSpec((1, KC), lambda j, i: (0, j)),
            pl.BlockSpec((KC, D), lambda j, i: (j, 0)),
        ],
        out_specs=[
            pl.BlockSpec((1, NN, B), lambda j, i: (j, 0, 0)),
            pl.BlockSpec((1, NN, B), lambda j, i: (j, 0, 0)),
        ],
        out_shape=[
            jax.ShapeDtypeStruct((NCH, NN, B), jnp.float32),
            jax.ShapeDtypeStruct((NCH, NN, B), jnp.int32),
        ],
        scratch_shapes=[pltpu.VMEM((BB, KC), jnp.float32),
                        pltpu.VMEM((BB, S), jnp.float32),
                        pltpu.VMEM((BB, S), jnp.int32)],
    )(x, q2, db2_pad, xn_pad)

    vals2 = jnp.concatenate(
        [vals_t.reshape(CAND, B),
         jnp.full((CPAD - CAND, B), jnp.inf, jnp.float32)], axis=0)
    idx2 = jnp.concatenate(
        [idx_t.reshape(CAND, B),
         jnp.zeros((CPAD - CAND, B), jnp.int32)], axis=0)
    vflat = vals2.reshape(CPAD, B // 16, 16).transpose(1, 0, 2).reshape(-1)
    iflat = idx2.reshape(CPAD, B // 16, 16).transpose(1, 0, 2).reshape(-1)
    preds = _make_stage2()(vflat, iflat, y_arr)
    return preds[:, None]


# speedup vs baseline: 4.4023x; 1.0936x over previous
import functools

import jax
import jax.numpy as jnp
from jax import lax
from jax.experimental import pallas as pl
from jax.experimental.pallas import tpu as pltpu
from jax.experimental.pallas import tpu_sc as plsc

B = 1024
D = 64
K = 100000
NN = 5
KC = 8704
NCH = 12
CAND = NCH * NN
CPAD = 64
NW = 32
QPW = B // NW


BB = 512
NB = B // BB
S = 128
NSTR = KC // S


def _stage1(x_ref, q2_ref, db2_ref, xn_ref, vals_ref, idx_ref, d_ref,
            best_ref, cbest_ref):
    j = pl.program_id(0)
    i = pl.program_id(1)
    dots = lax.dot_general(x_ref[...], xn_ref[...], (((1,), (1,)), ((), ())),
                           preferred_element_type=jnp.float32,
                           precision=lax.Precision.DEFAULT)
    d_ref[...] = (q2_ref[...] + db2_ref[...]) - 2.0 * dots
    big = jnp.int32(2 ** 30)
    inf = jnp.float32(jnp.inf)

    def pass_body(t, carry):
        m_prev, am_prev, mv, iv = carry
        mp = m_prev[:, None]
        ap = am_prev[:, None]
        best_ref[...] = jnp.full((BB, S), inf, jnp.float32)
        cbest_ref[...] = jnp.full((BB, S), big, jnp.int32)

        def strip(s, _):
            dc = d_ref[:, pl.ds(pl.multiple_of(s * S, S), S)]
            col = lax.broadcasted_iota(jnp.int32, (BB, S), 1) + s * S
            ok = (dc > mp) | ((dc == mp) & (col > ap))
            upd = ok & (dc < best_ref[...])
            best_ref[...] = jnp.where(upd, dc, best_ref[...])
            cbest_ref[...] = jnp.where(upd, col, cbest_ref[...])
            return 0

        lax.fori_loop(0, NSTR, strip, 0)
        best = best_ref[...]
        m = jnp.min(best, axis=1)
        am = jnp.min(jnp.where(best == m[:, None], cbest_ref[...], big),
                     axis=1)
        mv = tuple(jnp.where(t == tt, m, mv[tt]) for tt in range(NN))
        iv = tuple(jnp.where(t == tt, am, iv[tt]) for tt in range(NN))
        return m, am, mv, iv

    zf = jnp.zeros((BB,), jnp.float32)
    zi = jnp.zeros((BB,), jnp.int32)
    _, _, mv, iv = lax.fori_loop(
        0, NN, pass_body,
        (jnp.full((BB,), -inf, jnp.float32),
         jnp.full((BB,), jnp.int32(-1)), (zf,) * NN, (zi,) * NN))
    for tt in range(NN):
        vals_ref[0, tt, pl.ds(i * BB, BB)] = mv[tt]
        idx_ref[0, tt, pl.ds(i * BB, BB)] = iv[tt] + j * KC


def _stage2(vals_hbm, idx_hbm, y_hbm, out_hbm, vv, iv, yg, ov, sem):
    wid = lax.axis_index("s") * 2 + lax.axis_index("c")
    for g in range(QPW // 16):
        gi = wid * (QPW // 16) + g
        pltpu.sync_copy(vals_hbm.at[pl.ds(gi * CPAD * 16, CPAD * 16)], vv)
        pltpu.sync_copy(idx_hbm.at[pl.ds(gi * CPAD * 16, CPAD * 16)], iv)
        carry0 = (jnp.full((16,), jnp.inf, jnp.float32),) * NN + (
            jnp.zeros((16,), jnp.int32),) * NN

        def body(c, carry):
            bv = list(carry[:NN])
            bi = list(carry[NN:])
            v = vv[pl.ds(c * 16, 16)]
            ix = iv[pl.ds(c * 16, 16)]
            for t in range(NN):
                sw = v < bv[t]
                nv = jnp.where(sw, v, bv[t])
                ni = jnp.where(sw, ix, bi[t])
                v = jnp.where(sw, bv[t], v)
                ix = jnp.where(sw, bi[t], ix)
                bv[t] = nv
                bi[t] = ni
            return tuple(bv) + tuple(bi)

        carry = lax.fori_loop(0, CPAD, body, carry0)
        for t in range(NN):
            pltpu.async_copy(y_hbm.at[carry[NN + t]], yg.at[t], sem).wait()
        s = yg[0, :]
        for t in range(1, NN):
            s = s + yg[t, :]
        ov[...] = s * jnp.float32(1.0 / NN)
        pltpu.sync_copy(ov, out_hbm.at[pl.ds(gi * 16, 16)])


def _make_stage2():
    mesh = plsc.VectorSubcoreMesh(core_axis_name="c", subcore_axis_name="s")
    return pl.kernel(
        _stage2,
        mesh=mesh,
        out_type=jax.ShapeDtypeStruct((B,), jnp.float32),
        scratch_types=[
            pltpu.VMEM((CPAD * 16,), jnp.float32),
            pltpu.VMEM((CPAD * 16,), jnp.int32),
            pltpu.VMEM((NN, 16), jnp.float32),
            pltpu.VMEM((16,), jnp.float32),
            pltpu.SemaphoreType.DMA,
        ],
    )


def kernel(feats, feat_means, feat_stds, Xn, y_arr):
    x = (feats - feat_means) / feat_stds
    q2 = jnp.sum(x * x, axis=1, keepdims=True)
    db2 = jnp.sum(Xn * Xn, axis=1)[None, :]
    kpad = NCH * KC - K
    xn_pad = jnp.pad(Xn, ((0, kpad), (0, 0)))
    db2_pad = jnp.pad(db2, ((0, 0), (0, kpad)),
                      constant_values=jnp.inf)
    vals_t, idx_t = pl.pallas_call(
        _stage1,
        grid=(NCH, NB),
        in_specs=[
            pl.BlockSpec((BB, D), lambda j, i: (i, 0)),
            pl.BlockSpec((BB, 1), lambda j, i: (i, 0)),
            pl.BlockSpec((1, KC), lambda j, i: (0, j)),
            pl.BlockSpec((KC, D), lambda j, i: (j, 0)),
        ],
        out_specs=[
            pl.BlockSpec((1, NN, B), lambda j, i: (j, 0, 0)),
            pl.BlockSpec((1, NN, B), lambda j, i: (j, 0, 0)),
        ],
        out_shape=[
            jax.ShapeDtypeStruct((NCH, NN, B), jnp.float32),
            jax.ShapeDtypeStruct((NCH, NN, B), jnp.int32),
        ],
        scratch_shapes=[pltpu.VMEM((BB, KC), jnp.float32),
                        pltpu.VMEM((BB, S), jnp.float32),
                        pltpu.VMEM((BB, S), jnp.int32)],
    )(x, q2, db2_pad, xn_pad)

    vals2 = jnp.concatenate(
        [vals_t.reshape(CAND, B),
         jnp.full((CPAD - CAND, B), jnp.inf, jnp.float32)], axis=0)
    idx2 = jnp.concatenate(
        [idx_t.reshape(CAND, B),
         jnp.zeros((CPAD - CAND, B), jnp.int32)], axis=0)
    vflat = vals2.reshape(CPAD, B // 16, 16).transpose(1, 0, 2).reshape(-1)
    iflat = idx2.reshape(CPAD, B // 16, 16).transpose(1, 0, 2).reshape(-1)
    preds = _make_stage2()(vflat, iflat, y_arr)
    return preds[:, None]
